# Initial kernel scaffold; baseline (speedup 1.0000x reference)
import jax
import jax.numpy as jnp
from jax.experimental import pallas as pl


def kernel(task_features, edge_features, queue_edges, type_edges, affinity_edges, topology_edges, te_w1, te_b1, te_w2, te_b2, ee_w1, ee_b1, ee_w2, ee_b2, gnn_w0, gnn_b0, gnn_w1, gnn_b1, ta_w, ta_b, ea_w, ea_b, out_w1, out_b1, out_w2, out_b2):
    raise NotImplementedError("write your pallas kernel here")


# speedup vs baseline: 8.1961x; 8.1961x over previous
import functools

import jax
import jax.numpy as jnp
from jax import lax
from jax.experimental import pallas as pl
from jax.experimental.pallas import tpu as pltpu
from jax.experimental.pallas import tpu_sc as plsc

N_TASK = 50000
N_EDGE = 10000
HID = 64
HALF = 32
NSUB = 16
L = 16


def _rpt(n):
    r = -(-n // NSUB)
    return -(-r // 8) * 8


def _per_tile(s, n, fn):
    a = _rpt(n)
    b = n - (NSUB - 1) * a

    @pl.when(s < NSUB - 1)
    def _():
        fn(s * a, a)

    @pl.when(s == NSUB - 1)
    def _():
        fn(s * a, b)


def _fill_f32(ref, n, value):
    v = jnp.full((L,), value, jnp.float32)

    def body(i, _):
        ref[pl.ds(i * L, L)] = v
        return 0

    lax.fori_loop(0, n // L, body, 0)


def _zero_rows(ref, rows):
    z = jnp.zeros((L,), jnp.float32)

    def body(i, _):
        ref[i, pl.ds(0, L)] = z
        ref[i, pl.ds(L, L)] = z
        return 0

    lax.fori_loop(0, rows, body, 0)


def _copy_rows(src_ref, dst_ref, base, total):
    ch = src_ref.shape[0]
    nfull, rem = total // ch, total % ch
    for j in range(nfull):
        pltpu.sync_copy(src_ref.at[pl.ds(0, ch)],
                        dst_ref.at[pl.ds(base + j * ch, ch)])
    if rem:
        pltpu.sync_copy(src_ref.at[pl.ds(0, rem)],
                        dst_ref.at[pl.ds(base + nfull * ch, rem)])


NP_T = NSUB * _rpt(N_TASK)
NP_E = NSUB * _rpt(N_EDGE)


def _counts(tt, afs, aft, ot):
    mesh = plsc.VectorSubcoreMesh(core_axis_name="c", subcore_axis_name="s")
    f32 = jnp.float32
    CH = 2000

    @functools.partial(
        pl.kernel, mesh=mesh,
        compiler_params=pltpu.CompilerParams(use_tc_tiling_on_sc=False),
        out_type=[jax.ShapeDtypeStruct((NP_T,), f32),
                  jax.ShapeDtypeStruct((NP_T,), f32),
                  jax.ShapeDtypeStruct((NP_E,), f32),
                  jax.ShapeDtypeStruct((NP_E,), f32)],
        scratch_types=[
            pltpu.VMEM((CH,), jnp.int32),
            pltpu.VMEM((CH,), f32),
            pltpu.VMEM((CH,), f32),
            pltpu.VMEM_SHARED((NP_T,), f32),
            pltpu.VMEM_SHARED((NP_T,), f32),
            pltpu.VMEM_SHARED((NP_E,), f32),
            pltpu.VMEM_SHARED((NP_E,), f32),
        ])
    def ck(tt_h, afs_h, aft_h, ot_h, o_t, o_a, o_e, o_o,
           idx, ones, zeros, a_t, a_a, a_e, a_o):
        c = lax.axis_index("c")
        s = lax.axis_index("s")
        _fill_f32(ones, CH, 1.0)
        _fill_f32(zeros, CH, 0.0)
        for acc, rpt in ((a_t, _rpt(N_TASK)), (a_a, _rpt(N_TASK)),
                         (a_e, _rpt(N_EDGE)), (a_o, _rpt(N_EDGE))):
            _copy_rows(zeros, acc, s * rpt, rpt)
        plsc.subcore_barrier()

        def hist(src_h, acc, n_edges):
            per = n_edges // NSUB

            def body(k, _):
                pltpu.sync_copy(src_h.at[pl.ds(s * per + k * CH, CH)], idx)
                pltpu.sync_copy(ones, acc.at[idx], add=True)
                return 0

            lax.fori_loop(0, per // CH, body, 0)

        @pl.when(c == 0)
        def _():
            hist(tt_h, a_t, 800000)

        @pl.when(c == 1)
        def _():
            hist(afs_h, a_a, 160000)
            hist(aft_h, a_e, 160000)
            hist(ot_h, a_o, 160000)

        plsc.subcore_barrier()

        @pl.when(c == 0)
        def _():
            rpt = _rpt(N_TASK)
            pltpu.sync_copy(a_t.at[pl.ds(s * rpt, rpt)],
                            o_t.at[pl.ds(s * rpt, rpt)])

        @pl.when(c == 1)
        def _():
            rpt = _rpt(N_TASK)
            pltpu.sync_copy(a_a.at[pl.ds(s * rpt, rpt)],
                            o_a.at[pl.ds(s * rpt, rpt)])
            rpe = _rpt(N_EDGE)
            pltpu.sync_copy(a_e.at[pl.ds(s * rpe, rpe)],
                            o_e.at[pl.ds(s * rpe, rpe)])
            pltpu.sync_copy(a_o.at[pl.ds(s * rpe, rpe)],
                            o_o.at[pl.ds(s * rpe, rpe)])

    t, a, e, o = ck(tt, afs, aft, ot)
    return t[:N_TASK], a[:N_TASK], e[:N_EDGE], o[:N_EDGE]


_POOL_WORDS = 2_000_000


def _pick_ch(per, acc_words):
    for ch in (2000, 1000, 400, 200, 80, 40):
        if per % ch == 0 and acc_words + NSUB * 34 * ch <= _POOL_WORDS:
            return ch
    raise ValueError("no chunk size fits")


@functools.lru_cache(maxsize=None)
def _make_segsum(n_src, n_out, n_edges):
    mesh = plsc.VectorSubcoreMesh(core_axis_name="c", subcore_axis_name="s")
    f32 = jnp.float32
    per = n_edges // NSUB
    CH = _pick_ch(per, n_out * HALF)
    iters = per // CH

    @functools.partial(
        pl.kernel, mesh=mesh,
        compiler_params=pltpu.CompilerParams(use_tc_tiling_on_sc=False),
        out_type=[jax.ShapeDtypeStruct((n_out, HALF), f32),
                  jax.ShapeDtypeStruct((n_out, HALF), f32)],
        scratch_types=[
            pltpu.VMEM((CH,), jnp.int32),
            pltpu.VMEM((CH,), jnp.int32),
            pltpu.VMEM((CH, HALF), f32),
            pltpu.VMEM_SHARED((n_out, HALF), f32),
            pltpu.SemaphoreType.DMA,
        ])
    def k(tlo, thi, src, dst, out_lo, out_hi, sidx, didx, rows, acc, sem):
        c = lax.axis_index("c")
        s = lax.axis_index("s")
        _zero_rows(rows, CH)
        _per_tile(s, n_out, lambda base, sz: _copy_rows(rows, acc, base, sz))
        plsc.subcore_barrier()

        def body(kk, _):
            off = s * per + kk * CH
            pltpu.sync_copy(src.at[pl.ds(off, CH)], sidx)
            pltpu.sync_copy(dst.at[pl.ds(off, CH)], didx)

            @pl.when(c == 0)
            def _():
                pltpu.async_copy(tlo.at[sidx], rows, sem).wait()

            @pl.when(c == 1)
            def _():
                pltpu.async_copy(thi.at[sidx], rows, sem).wait()

            pltpu.sync_copy(rows, acc.at[didx], add=True)
            return 0

        lax.fori_loop(0, iters, body, 0)
        plsc.subcore_barrier()

        def wr(out_ref):
            return lambda base, sz: pltpu.sync_copy(
                acc.at[pl.ds(base, sz)], out_ref.at[pl.ds(base, sz)])

        @pl.when(c == 0)
        def _():
            _per_tile(s, n_out, wr(out_lo))

        @pl.when(c == 1)
        def _():
            _per_tile(s, n_out, wr(out_hi))

    return k


_BR = 1000


def _dotf(a, b):
    return jnp.dot(a, b, preferred_element_type=jnp.float32)


def _enc_body(x, w1, b1, w2, b2, lo, hi):
    h = jnp.maximum(_dotf(x[...], w1[...]) + b1[...], 0.0)
    y = jnp.maximum(_dotf(h, w2[...]) + b2[...], 0.0)
    lo[...] = y[:, :HALF]
    hi[...] = y[:, HALF:]


def _encode(x, w1, b1, w2, b2):
    n = x.shape[0]
    f32 = jnp.float32
    full = lambda shp: pl.BlockSpec(shp, lambda i: (0, 0))
    return pl.pallas_call(
        _enc_body,
        grid=(n // _BR,),
        in_specs=[pl.BlockSpec((_BR, 8), lambda i: (i, 0)),
                  full((8, HID)), full((1, HID)),
                  full((HID, HID)), full((1, HID))],
        out_specs=[pl.BlockSpec((_BR, HALF), lambda i: (i, 0))] * 2,
        out_shape=[jax.ShapeDtypeStruct((n, HALF), f32)] * 2,
    )(x, w1, b1, w2, b2)


def _gnn_body(tlo, thi, qlo, qhi, w0, b0, w1, b1, olo, ohi):
    t = jnp.concatenate([tlo[...], thi[...]], axis=1)
    q = jnp.concatenate([qlo[...], qhi[...]], axis=1) * 0.5
    h = jnp.maximum(_dotf(t + q, w0[...]) + b0[...], 0.0)
    h = jnp.maximum(_dotf(h + q, w1[...]) + b1[...], 0.0)
    olo[...] = h[:, :HALF]
    ohi[...] = h[:, HALF:]


def _gnn(tlo, thi, qlo, qhi, w0, b0, w1, b1):
    f32 = jnp.float32
    full = lambda shp: pl.BlockSpec(shp, lambda i: (0, 0))
    half = pl.BlockSpec((_BR, HALF), lambda i: (i, 0))
    return pl.pallas_call(
        _gnn_body,
        grid=(N_TASK // _BR,),
        in_specs=[half, half, half, half,
                  full((HID, HID)), full((1, HID)),
                  full((HID, HID)), full((1, HID))],
        out_specs=[half, half],
        out_shape=[jax.ShapeDtypeStruct((N_TASK, HALF), f32)] * 2,
    )(tlo, thi, qlo, qhi, w0, b0, w1, b1)


def _upd_body(hlo, hhi, mlo, mhi, cnt, olo, ohi, ssum):
    h = jnp.concatenate([hlo[...], hhi[...]], axis=1)
    m = jnp.concatenate([mlo[...], mhi[...]], axis=1)
    h2 = h + (m / jnp.maximum(cnt[...], 1.0)) * 0.3
    olo[...] = h2[:, :HALF]
    ohi[...] = h2[:, HALF:]

    @pl.when(pl.program_id(0) == 0)
    def _():
        ssum[...] = jnp.zeros_like(ssum)

    ssum[...] += jnp.sum(h2, axis=0, keepdims=True)


def _update(hlo, hhi, mlo, mhi, cnt):
    n = hlo.shape[0]
    f32 = jnp.float32
    half = pl.BlockSpec((_BR, HALF), lambda i: (i, 0))
    return pl.pallas_call(
        _upd_body,
        grid=(n // _BR,),
        in_specs=[half, half, half, half,
                  pl.BlockSpec((_BR, 1), lambda i: (i, 0))],
        out_specs=[half, half, pl.BlockSpec((1, HID), lambda i: (0, 0))],
        out_shape=[jax.ShapeDtypeStruct((n, HALF), f32)] * 2
        + [jax.ShapeDtypeStruct((1, HID), f32)],
    )(hlo, hhi, mlo, mhi, cnt)


def _msum_body(mlo, mhi, cnt, ssum):
    m = jnp.concatenate([mlo[...], mhi[...]], axis=1)
    term = (m / jnp.maximum(cnt[...], 1.0)) * 0.3

    @pl.when(pl.program_id(0) == 0)
    def _():
        ssum[...] = jnp.zeros_like(ssum)

    ssum[...] += jnp.sum(term, axis=0, keepdims=True)


def _msum(mlo, mhi, cnt):
    n = mlo.shape[0]
    half = pl.BlockSpec((_BR, HALF), lambda i: (i, 0))
    return pl.pallas_call(
        _msum_body,
        grid=(n // _BR,),
        in_specs=[half, half, pl.BlockSpec((_BR, 1), lambda i: (i, 0))],
        out_specs=[pl.BlockSpec((1, HID), lambda i: (0, 0))],
        out_shape=[jax.ShapeDtypeStruct((1, HID), jnp.float32)],
    )(mlo, mhi, cnt)[0]


def _final_body(omlo, omhi, ocnt, hsum1, hsum2, esum,
                ta_w, ta_b, ea_w, ea_b, ow1, ob1, ow2, ob2, out):
    om = jnp.concatenate([omlo[...], omhi[...]], axis=1)
    osum = jnp.sum(om / jnp.maximum(ocnt[...], 1.0), axis=0, keepdims=True) * 0.3
    hmean = (hsum1[...] + hsum2[...]) / N_TASK
    emean = (esum[...] + osum) / N_EDGE
    t_agg = jnp.maximum(_dotf(hmean, ta_w[...]) + ta_b[...], 0.0)
    e_agg = jnp.maximum(_dotf(emean, ea_w[...]) + ea_b[...], 0.0)
    comb = jnp.concatenate([t_agg, e_agg], axis=1)
    y = jnp.maximum(_dotf(comb, ow1[...]) + ob1[...], 0.0)
    out[...] = _dotf(y, ow2[...]) + ob2[...]


def _final(omlo, omhi, ocnt, hsum1, hsum2, esum,
           ta_w, ta_b, ea_w, ea_b, ow1, ob1, ow2, ob2):
    f32 = jnp.float32
    full = lambda shp: pl.BlockSpec(shp, lambda: tuple(0 for _ in shp))
    args = (omlo, omhi, ocnt, hsum1, hsum2, esum,
            ta_w, ta_b, ea_w, ea_b, ow1, ob1, ow2, ob2)
    return pl.pallas_call(
        _final_body,
        in_specs=[full(a.shape) for a in args],
        out_specs=full((1, HID)),
        out_shape=jax.ShapeDtypeStruct((1, HID), f32),
    )(*args)


def kernel(task_features, edge_features, queue_edges, type_edges,
           affinity_edges, topology_edges,
           te_w1, te_b1, te_w2, te_b2, ee_w1, ee_b1, ee_w2, ee_b2,
           gnn_w0, gnn_b0, gnn_w1, gnn_b1, ta_w, ta_b, ea_w, ea_b,
           out_w1, out_b1, out_w2, out_b2):
    r1 = lambda b: b.reshape(1, -1)
    xt = jnp.pad(task_features, ((0, 0), (0, 2)))
    xe = jnp.pad(edge_features, ((0, 0), (0, 2)))
    tw1 = jnp.pad(te_w1, ((0, 2), (0, 0)))
    ew1 = jnp.pad(ee_w1, ((0, 2), (0, 0)))
    qs, qt = queue_edges[0], queue_edges[1]
    ts, tt = type_edges[0], type_edges[1]
    afs, aft = affinity_edges[0], affinity_edges[1]
    os_, ot = topology_edges[0], topology_edges[1]

    tcnt, acnt, ecnt, ocnt = _counts(tt, afs, aft, ot)

    t_lo, t_hi = _encode(xt, tw1, r1(te_b1), te_w2, r1(te_b2))
    e_lo, e_hi = _encode(xe, ew1, r1(ee_b1), ee_w2, r1(ee_b2))

    q_lo, q_hi = _make_segsum(N_TASK, N_TASK, 800000)(t_lo, t_hi, qs, qt)
    h_lo, h_hi = _gnn(t_lo, t_hi, q_lo, q_hi,
                      gnn_w0, r1(gnn_b0), gnn_w1, r1(gnn_b1))

    tm_lo, tm_hi = _make_segsum(N_TASK, N_TASK, 800000)(h_lo, h_hi, ts, tt)
    h2_lo, h2_hi, h2sum = _update(h_lo, h_hi, tm_lo, tm_hi,
                                  tcnt.reshape(-1, 1))

    am_lo, am_hi = _make_segsum(N_EDGE, N_TASK, 160000)(e_lo, e_hi, aft, afs)
    em_lo, em_hi = _make_segsum(N_TASK, N_EDGE, 160000)(h2_lo, h2_hi, afs, aft)
    tasksum = _msum(am_lo, am_hi, acnt.reshape(-1, 1))
    e2_lo, e2_hi, e2sum = _update(e_lo, e_hi, em_lo, em_hi,
                                  ecnt.reshape(-1, 1))

    om_lo, om_hi = _make_segsum(N_EDGE, N_EDGE, 160000)(e2_lo, e2_hi, os_, ot)

    out = _final(om_lo, om_hi, ocnt.reshape(-1, 1), h2sum, tasksum, e2sum,
                 ta_w, r1(ta_b), ea_w, r1(ea_b),
                 out_w1, r1(out_b1), out_w2, r1(out_b2))
    return out.reshape(HID)


# speedup vs baseline: 11.4211x; 1.3935x over previous
import functools

import jax
import jax.numpy as jnp
from jax import lax
from jax.experimental import pallas as pl
from jax.experimental.pallas import tpu as pltpu
from jax.experimental.pallas import tpu_sc as plsc

N_TASK = 50000
N_EDGE = 10000
HID = 64
HALF = 32
NSUB = 16
L = 16


def _rpt(n):
    r = -(-n // NSUB)
    return -(-r // 8) * 8


def _per_tile(s, n, fn):
    a = _rpt(n)
    b = n - (NSUB - 1) * a

    @pl.when(s < NSUB - 1)
    def _():
        fn(s * a, a)

    @pl.when(s == NSUB - 1)
    def _():
        fn(s * a, b)


def _fill_f32(ref, n, value):
    v = jnp.full((L,), value, jnp.float32)

    def body(i, _):
        ref[pl.ds(i * L, L)] = v
        return 0

    lax.fori_loop(0, n // L, body, 0)


def _zero_rows(ref, rows):
    z = jnp.zeros((L,), jnp.float32)

    def body(i, _):
        ref[i, pl.ds(0, L)] = z
        ref[i, pl.ds(L, L)] = z
        return 0

    lax.fori_loop(0, rows, body, 0)


def _copy_rows(src_ref, dst_ref, base, total):
    ch = src_ref.shape[0]
    nfull, rem = total // ch, total % ch
    for j in range(nfull):
        pltpu.sync_copy(src_ref.at[pl.ds(0, ch)],
                        dst_ref.at[pl.ds(base + j * ch, ch)])
    if rem:
        pltpu.sync_copy(src_ref.at[pl.ds(0, rem)],
                        dst_ref.at[pl.ds(base + nfull * ch, rem)])


NP_T = NSUB * _rpt(N_TASK)
NP_E = NSUB * _rpt(N_EDGE)


def _counts(tt, afs, aft, ot):
    mesh = plsc.VectorSubcoreMesh(core_axis_name="c", subcore_axis_name="s")
    f32 = jnp.float32
    CH = 2000

    @functools.partial(
        pl.kernel, mesh=mesh,
        compiler_params=pltpu.CompilerParams(use_tc_tiling_on_sc=False),
        out_type=[jax.ShapeDtypeStruct((NP_T,), f32),
                  jax.ShapeDtypeStruct((NP_T,), f32),
                  jax.ShapeDtypeStruct((NP_E,), f32),
                  jax.ShapeDtypeStruct((NP_E,), f32)],
        scratch_types=[
            pltpu.VMEM((CH,), jnp.int32),
            pltpu.VMEM((CH,), f32),
            pltpu.VMEM((CH,), f32),
            pltpu.VMEM_SHARED((NP_T,), f32),
            pltpu.VMEM_SHARED((NP_T,), f32),
            pltpu.VMEM_SHARED((NP_E,), f32),
            pltpu.VMEM_SHARED((NP_E,), f32),
        ])
    def ck(tt_h, afs_h, aft_h, ot_h, o_t, o_a, o_e, o_o,
           idx, ones, zeros, a_t, a_a, a_e, a_o):
        c = lax.axis_index("c")
        s = lax.axis_index("s")
        _fill_f32(ones, CH, 1.0)
        _fill_f32(zeros, CH, 0.0)
        for acc, rpt in ((a_t, _rpt(N_TASK)), (a_a, _rpt(N_TASK)),
                         (a_e, _rpt(N_EDGE)), (a_o, _rpt(N_EDGE))):
            _copy_rows(zeros, acc, s * rpt, rpt)
        plsc.subcore_barrier()

        def hist(src_h, acc, n_edges):
            per = n_edges // NSUB

            def body(k, _):
                pltpu.sync_copy(src_h.at[pl.ds(s * per + k * CH, CH)], idx)
                pltpu.sync_copy(ones, acc.at[idx], add=True)
                return 0

            lax.fori_loop(0, per // CH, body, 0)

        @pl.when(c == 0)
        def _():
            hist(tt_h, a_t, 800000)

        @pl.when(c == 1)
        def _():
            hist(afs_h, a_a, 160000)
            hist(aft_h, a_e, 160000)
            hist(ot_h, a_o, 160000)

        plsc.subcore_barrier()

        @pl.when(c == 0)
        def _():
            rpt = _rpt(N_TASK)
            pltpu.sync_copy(a_t.at[pl.ds(s * rpt, rpt)],
                            o_t.at[pl.ds(s * rpt, rpt)])

        @pl.when(c == 1)
        def _():
            rpt = _rpt(N_TASK)
            pltpu.sync_copy(a_a.at[pl.ds(s * rpt, rpt)],
                            o_a.at[pl.ds(s * rpt, rpt)])
            rpe = _rpt(N_EDGE)
            pltpu.sync_copy(a_e.at[pl.ds(s * rpe, rpe)],
                            o_e.at[pl.ds(s * rpe, rpe)])
            pltpu.sync_copy(a_o.at[pl.ds(s * rpe, rpe)],
                            o_o.at[pl.ds(s * rpe, rpe)])

    t, a, e, o = ck(tt, afs, aft, ot)
    return t[:N_TASK], a[:N_TASK], e[:N_EDGE], o[:N_EDGE]


_POOL_WORDS = 2_090_000


def _pick_ch(per, acc_words):
    for ch in (2000, 1000, 400, 200, 80, 40):
        if per % ch == 0 and acc_words + NSUB * 68 * ch <= _POOL_WORDS:
            return ch
    raise ValueError("no chunk size fits")


@functools.lru_cache(maxsize=None)
def _make_segsum(n_src, n_out, n_edges):
    mesh = plsc.VectorSubcoreMesh(core_axis_name="c", subcore_axis_name="s")
    f32 = jnp.float32
    per = n_edges // NSUB
    CH = _pick_ch(per, n_out * HALF)
    iters = per // CH
    pairs, tail = iters // 2, iters % 2

    @functools.partial(
        pl.kernel, mesh=mesh,
        compiler_params=pltpu.CompilerParams(use_tc_tiling_on_sc=False),
        out_type=[jax.ShapeDtypeStruct((n_out, HALF), f32),
                  jax.ShapeDtypeStruct((n_out, HALF), f32)],
        scratch_types=[
            pltpu.VMEM((CH,), jnp.int32), pltpu.VMEM((CH,), jnp.int32),
            pltpu.VMEM((CH, HALF), f32),
            pltpu.VMEM((CH,), jnp.int32), pltpu.VMEM((CH,), jnp.int32),
            pltpu.VMEM((CH, HALF), f32),
            pltpu.VMEM_SHARED((n_out, HALF), f32),
        ] + [pltpu.SemaphoreType.DMA] * 6)
    def k(tlo, thi, src, dst, out_lo, out_hi,
          sidx0, didx0, rows0, sidx1, didx1, rows1, acc,
          si0, sg0, ss0, si1, sg1, ss1):
        c = lax.axis_index("c")
        s = lax.axis_index("s")
        _zero_rows(rows0, CH)
        _per_tile(s, n_out, lambda base, sz: _copy_rows(rows0, acc, base, sz))
        plsc.subcore_barrier()
        ebase = s * per

        bufs = ((sidx0, didx0, rows0, si0, sg0, ss0),
                (sidx1, didx1, rows1, si1, sg1, ss1))

        def start_idx(b, off):
            sidx, didx, _, si, _, _ = bufs[b]
            pltpu.make_async_copy(src.at[pl.ds(off, CH)], sidx, si).start()
            pltpu.make_async_copy(dst.at[pl.ds(off, CH)], didx, si).start()

        def start_gather(b, off):
            sidx, didx, rows, si, sg, _ = bufs[b]
            pltpu.make_async_copy(src.at[pl.ds(off, CH)], sidx, si).wait()
            pltpu.make_async_copy(dst.at[pl.ds(off, CH)], didx, si).wait()

            @pl.when(c == 0)
            def _():
                pltpu.make_async_copy(tlo.at[sidx], rows, sg).start()

            @pl.when(c == 1)
            def _():
                pltpu.make_async_copy(thi.at[sidx], rows, sg).start()

        def start_scatter(b):
            sidx, didx, rows, _, sg, ss = bufs[b]

            @pl.when(c == 0)
            def _():
                pltpu.make_async_copy(tlo.at[sidx], rows, sg).wait()

            @pl.when(c == 1)
            def _():
                pltpu.make_async_copy(thi.at[sidx], rows, sg).wait()

            pltpu.make_async_copy(rows, acc.at[didx], ss).start(add=True)

        def wait_scatter(b):
            _, didx, rows, _, _, ss = bufs[b]
            pltpu.make_async_copy(rows, acc.at[didx], ss).wait()

        def pair(g, _):
            off0 = ebase + (2 * g) * CH
            off1 = off0 + CH

            @pl.when(g > 0)
            def _():
                wait_scatter(0)

            start_idx(0, off0)
            start_gather(0, off0)

            @pl.when(g > 0)
            def _():
                wait_scatter(1)

            start_idx(1, off1)
            start_gather(1, off1)
            start_scatter(0)
            start_scatter(1)
            return 0

        if pairs:
            lax.fori_loop(0, pairs, pair, 0)
        if tail:
            off = ebase + 2 * pairs * CH
            if pairs:
                wait_scatter(0)
            start_idx(0, off)
            start_gather(0, off)
            start_scatter(0)
        if pairs or tail:
            wait_scatter(0)
        if pairs:
            wait_scatter(1)
        plsc.subcore_barrier()

        def wr(out_ref):
            return lambda base, sz: pltpu.sync_copy(
                acc.at[pl.ds(base, sz)], out_ref.at[pl.ds(base, sz)])

        @pl.when(c == 0)
        def _():
            _per_tile(s, n_out, wr(out_lo))

        @pl.when(c == 1)
        def _():
            _per_tile(s, n_out, wr(out_hi))

    return k


_BR = 1000


def _dotf(a, b):
    return jnp.dot(a, b, preferred_element_type=jnp.float32)


def _enc_body(x, w1, b1, w2, b2, lo, hi):
    h = jnp.maximum(_dotf(x[...], w1[...]) + b1[...], 0.0)
    y = jnp.maximum(_dotf(h, w2[...]) + b2[...], 0.0)
    lo[...] = y[:, :HALF]
    hi[...] = y[:, HALF:]


def _encode(x, w1, b1, w2, b2):
    n = x.shape[0]
    f32 = jnp.float32
    full = lambda shp: pl.BlockSpec(shp, lambda i: (0, 0))
    return pl.pallas_call(
        _enc_body,
        grid=(n // _BR,),
        in_specs=[pl.BlockSpec((_BR, 8), lambda i: (i, 0)),
                  full((8, HID)), full((1, HID)),
                  full((HID, HID)), full((1, HID))],
        out_specs=[pl.BlockSpec((_BR, HALF), lambda i: (i, 0))] * 2,
        out_shape=[jax.ShapeDtypeStruct((n, HALF), f32)] * 2,
    )(x, w1, b1, w2, b2)


def _gnn_body(tlo, thi, qlo, qhi, w0, b0, w1, b1, olo, ohi):
    t = jnp.concatenate([tlo[...], thi[...]], axis=1)
    q = jnp.concatenate([qlo[...], qhi[...]], axis=1) * 0.5
    h = jnp.maximum(_dotf(t + q, w0[...]) + b0[...], 0.0)
    h = jnp.maximum(_dotf(h + q, w1[...]) + b1[...], 0.0)
    olo[...] = h[:, :HALF]
    ohi[...] = h[:, HALF:]


def _gnn(tlo, thi, qlo, qhi, w0, b0, w1, b1):
    f32 = jnp.float32
    full = lambda shp: pl.BlockSpec(shp, lambda i: (0, 0))
    half = pl.BlockSpec((_BR, HALF), lambda i: (i, 0))
    return pl.pallas_call(
        _gnn_body,
        grid=(N_TASK // _BR,),
        in_specs=[half, half, half, half,
                  full((HID, HID)), full((1, HID)),
                  full((HID, HID)), full((1, HID))],
        out_specs=[half, half],
        out_shape=[jax.ShapeDtypeStruct((N_TASK, HALF), f32)] * 2,
    )(tlo, thi, qlo, qhi, w0, b0, w1, b1)


def _upd_body(hlo, hhi, mlo, mhi, cnt, olo, ohi, ssum):
    h = jnp.concatenate([hlo[...], hhi[...]], axis=1)
    m = jnp.concatenate([mlo[...], mhi[...]], axis=1)
    h2 = h + (m / jnp.maximum(cnt[...], 1.0)) * 0.3
    olo[...] = h2[:, :HALF]
    ohi[...] = h2[:, HALF:]

    @pl.when(pl.program_id(0) == 0)
    def _():
        ssum[...] = jnp.zeros_like(ssum)

    ssum[...] += jnp.sum(h2, axis=0, keepdims=True)


def _update(hlo, hhi, mlo, mhi, cnt):
    n = hlo.shape[0]
    f32 = jnp.float32
    half = pl.BlockSpec((_BR, HALF), lambda i: (i, 0))
    return pl.pallas_call(
        _upd_body,
        grid=(n // _BR,),
        in_specs=[half, half, half, half,
                  pl.BlockSpec((_BR, 1), lambda i: (i, 0))],
        out_specs=[half, half, pl.BlockSpec((1, HID), lambda i: (0, 0))],
        out_shape=[jax.ShapeDtypeStruct((n, HALF), f32)] * 2
        + [jax.ShapeDtypeStruct((1, HID), f32)],
    )(hlo, hhi, mlo, mhi, cnt)


def _msum_body(mlo, mhi, cnt, ssum):
    m = jnp.concatenate([mlo[...], mhi[...]], axis=1)
    term = (m / jnp.maximum(cnt[...], 1.0)) * 0.3

    @pl.when(pl.program_id(0) == 0)
    def _():
        ssum[...] = jnp.zeros_like(ssum)

    ssum[...] += jnp.sum(term, axis=0, keepdims=True)


def _msum(mlo, mhi, cnt):
    n = mlo.shape[0]
    half = pl.BlockSpec((_BR, HALF), lambda i: (i, 0))
    return pl.pallas_call(
        _msum_body,
        grid=(n // _BR,),
        in_specs=[half, half, pl.BlockSpec((_BR, 1), lambda i: (i, 0))],
        out_specs=[pl.BlockSpec((1, HID), lambda i: (0, 0))],
        out_shape=[jax.ShapeDtypeStruct((1, HID), jnp.float32)],
    )(mlo, mhi, cnt)[0]


def _final_body(omlo, omhi, ocnt, hsum1, hsum2, esum,
                ta_w, ta_b, ea_w, ea_b, ow1, ob1, ow2, ob2, out):
    om = jnp.concatenate([omlo[...], omhi[...]], axis=1)
    osum = jnp.sum(om / jnp.maximum(ocnt[...], 1.0), axis=0, keepdims=True) * 0.3
    hmean = (hsum1[...] + hsum2[...]) / N_TASK
    emean = (esum[...] + osum) / N_EDGE
    t_agg = jnp.maximum(_dotf(hmean, ta_w[...]) + ta_b[...], 0.0)
    e_agg = jnp.maximum(_dotf(emean, ea_w[...]) + ea_b[...], 0.0)
    comb = jnp.concatenate([t_agg, e_agg], axis=1)
    y = jnp.maximum(_dotf(comb, ow1[...]) + ob1[...], 0.0)
    out[...] = _dotf(y, ow2[...]) + ob2[...]


def _final(omlo, omhi, ocnt, hsum1, hsum2, esum,
           ta_w, ta_b, ea_w, ea_b, ow1, ob1, ow2, ob2):
    f32 = jnp.float32
    full = lambda shp: pl.BlockSpec(shp, lambda: tuple(0 for _ in shp))
    args = (omlo, omhi, ocnt, hsum1, hsum2, esum,
            ta_w, ta_b, ea_w, ea_b, ow1, ob1, ow2, ob2)
    return pl.pallas_call(
        _final_body,
        in_specs=[full(a.shape) for a in args],
        out_specs=full((1, HID)),
        out_shape=jax.ShapeDtypeStruct((1, HID), f32),
    )(*args)


def kernel(task_features, edge_features, queue_edges, type_edges,
           affinity_edges, topology_edges,
           te_w1, te_b1, te_w2, te_b2, ee_w1, ee_b1, ee_w2, ee_b2,
           gnn_w0, gnn_b0, gnn_w1, gnn_b1, ta_w, ta_b, ea_w, ea_b,
           out_w1, out_b1, out_w2, out_b2):
    r1 = lambda b: b.reshape(1, -1)
    xt = jnp.pad(task_features, ((0, 0), (0, 2)))
    xe = jnp.pad(edge_features, ((0, 0), (0, 2)))
    tw1 = jnp.pad(te_w1, ((0, 2), (0, 0)))
    ew1 = jnp.pad(ee_w1, ((0, 2), (0, 0)))
    qs, qt = queue_edges[0], queue_edges[1]
    ts, tt = type_edges[0], type_edges[1]
    afs, aft = affinity_edges[0], affinity_edges[1]
    os_, ot = topology_edges[0], topology_edges[1]

    tcnt, acnt, ecnt, ocnt = _counts(tt, afs, aft, ot)

    t_lo, t_hi = _encode(xt, tw1, r1(te_b1), te_w2, r1(te_b2))
    e_lo, e_hi = _encode(xe, ew1, r1(ee_b1), ee_w2, r1(ee_b2))

    q_lo, q_hi = _make_segsum(N_TASK, N_TASK, 800000)(t_lo, t_hi, qs, qt)
    h_lo, h_hi = _gnn(t_lo, t_hi, q_lo, q_hi,
                      gnn_w0, r1(gnn_b0), gnn_w1, r1(gnn_b1))

    tm_lo, tm_hi = _make_segsum(N_TASK, N_TASK, 800000)(h_lo, h_hi, ts, tt)
    h2_lo, h2_hi, h2sum = _update(h_lo, h_hi, tm_lo, tm_hi,
                                  tcnt.reshape(-1, 1))

    am_lo, am_hi = _make_segsum(N_EDGE, N_TASK, 160000)(e_lo, e_hi, aft, afs)
    em_lo, em_hi = _make_segsum(N_TASK, N_EDGE, 160000)(h2_lo, h2_hi, afs, aft)
    tasksum = _msum(am_lo, am_hi, acnt.reshape(-1, 1))
    e2_lo, e2_hi, e2sum = _update(e_lo, e_hi, em_lo, em_hi,
                                  ecnt.reshape(-1, 1))

    om_lo, om_hi = _make_segsum(N_EDGE, N_EDGE, 160000)(e2_lo, e2_hi, os_, ot)

    out = _final(om_lo, om_hi, ocnt.reshape(-1, 1), h2sum, tasksum, e2sum,
                 ta_w, r1(ta_b), ea_w, r1(ea_b),
                 out_w1, r1(out_b1), out_w2, r1(out_b2))
    return out.reshape(HID)


# speedup vs baseline: 16.2455x; 1.4224x over previous
import functools

import jax
import jax.numpy as jnp
from jax import lax
from jax.experimental import pallas as pl
from jax.experimental.pallas import tpu as pltpu
from jax.experimental.pallas import tpu_sc as plsc

N_TASK = 50000
N_EDGE = 10000
NT_P = 51200
NE_P = 10240
HID = 64
HALF = 32
NSUB = 16
L = 16
EQ = 800000
EA = 160000


def _fill_rows(ref, rows, value):
    v = jnp.full((L,), value, jnp.float32)

    def body(i, _):
        ref[i, pl.ds(0, L)] = v
        ref[i, pl.ds(L, L)] = v
        return 0

    lax.fori_loop(0, rows, body, 0)


def _copy_rows(src_ref, dst_ref, base, total):
    ch = src_ref.shape[0]
    nfull, rem = total // ch, total % ch
    for j in range(nfull):
        pltpu.sync_copy(src_ref.at[pl.ds(0, ch)],
                        dst_ref.at[pl.ds(base + j * ch, ch)])
    if rem:
        pltpu.sync_copy(src_ref.at[pl.ds(0, rem)],
                        dst_ref.at[pl.ds(base + nfull * ch, rem)])


_POOL_WORDS = 2_090_000


def _pick_ch(per, acc_words, words_per_ch):
    for ch in (2000, 1000, 400, 200, 80, 40):
        if per % ch == 0 and acc_words + NSUB * words_per_ch * ch <= _POOL_WORDS:
            return ch
    raise ValueError("no chunk size fits")


@functools.lru_cache(maxsize=None)
def _make_segsum(n_src, n_out, n_edges, src_first=True):
    mesh = plsc.VectorSubcoreMesh(core_axis_name="c", subcore_axis_name="s")
    f32 = jnp.float32
    per = n_edges // NSUB
    CH = _pick_ch(per, n_out * HALF, 68)
    iters = per // CH
    pairs, tail = iters // 2, iters % 2
    rows_t = n_out // NSUB
    s_off = 0 if src_first else n_edges
    d_off = n_edges - s_off

    @functools.partial(
        pl.kernel, mesh=mesh,
        compiler_params=pltpu.CompilerParams(use_tc_tiling_on_sc=False),
        out_type=[jax.ShapeDtypeStruct((n_out, HALF), f32),
                  jax.ShapeDtypeStruct((n_out, HALF), f32)],
        scratch_types=[
            pltpu.VMEM((CH,), jnp.int32), pltpu.VMEM((CH,), jnp.int32),
            pltpu.VMEM((CH, HALF), f32),
            pltpu.VMEM((CH,), jnp.int32), pltpu.VMEM((CH,), jnp.int32),
            pltpu.VMEM((CH, HALF), f32),
            pltpu.VMEM_SHARED((n_out, HALF), f32),
        ] + [pltpu.SemaphoreType.DMA] * 6)
    def k(tlo, thi, edges, out_lo, out_hi,
          sidx0, didx0, rows0, sidx1, didx1, rows1, acc,
          si0, sg0, ss0, si1, sg1, ss1):
        c = lax.axis_index("c")
        s = lax.axis_index("s")
        _fill_rows(rows0, CH, 0.0)
        _copy_rows(rows0, acc, s * rows_t, rows_t)
        plsc.subcore_barrier()
        ebase = s * per

        bufs = ((sidx0, didx0, rows0, si0, sg0, ss0),
                (sidx1, didx1, rows1, si1, sg1, ss1))

        def start_idx(b, off):
            sidx, didx, _, si, _, _ = bufs[b]
            pltpu.make_async_copy(
                edges.at[pl.ds(s_off + off, CH)], sidx, si).start()
            pltpu.make_async_copy(
                edges.at[pl.ds(d_off + off, CH)], didx, si).start()

        def start_gather(b, off):
            sidx, didx, rows, si, sg, _ = bufs[b]
            pltpu.make_async_copy(
                edges.at[pl.ds(s_off + off, CH)], sidx, si).wait()
            pltpu.make_async_copy(
                edges.at[pl.ds(d_off + off, CH)], didx, si).wait()

            @pl.when(c == 0)
            def _():
                pltpu.make_async_copy(tlo.at[sidx], rows, sg).start()

            @pl.when(c == 1)
            def _():
                pltpu.make_async_copy(thi.at[sidx], rows, sg).start()

        def start_scatter(b):
            sidx, didx, rows, _, sg, ss = bufs[b]

            @pl.when(c == 0)
            def _():
                pltpu.make_async_copy(tlo.at[sidx], rows, sg).wait()

            @pl.when(c == 1)
            def _():
                pltpu.make_async_copy(thi.at[sidx], rows, sg).wait()

            pltpu.make_async_copy(rows, acc.at[didx], ss).start(add=True)

        def wait_scatter(b):
            _, didx, rows, _, _, ss = bufs[b]
            pltpu.make_async_copy(rows, acc.at[didx], ss).wait()

        def pair(g, _):
            off0 = ebase + (2 * g) * CH
            off1 = off0 + CH

            @pl.when(g > 0)
            def _():
                wait_scatter(0)

            start_idx(0, off0)
            start_gather(0, off0)

            @pl.when(g > 0)
            def _():
                wait_scatter(1)

            start_idx(1, off1)
            start_gather(1, off1)
            start_scatter(0)
            start_scatter(1)
            return 0

        if pairs:
            lax.fori_loop(0, pairs, pair, 0)
        if tail:
            off = ebase + 2 * pairs * CH
            if pairs:
                wait_scatter(0)
            start_idx(0, off)
            start_gather(0, off)
            start_scatter(0)
        if pairs or tail:
            wait_scatter(0)
        if pairs:
            wait_scatter(1)
        plsc.subcore_barrier()

        def wr(out_ref):
            pltpu.sync_copy(acc.at[pl.ds(s * rows_t, rows_t)],
                            out_ref.at[pl.ds(s * rows_t, rows_t)])

        @pl.when(c == 0)
        def _():
            wr(out_lo)

        @pl.when(c == 1)
        def _():
            wr(out_hi)

    return k


@functools.lru_cache(maxsize=None)
def _make_ones_scatter(n0, e0, d_off0, n1, e1, d_off1):
    mesh = plsc.VectorSubcoreMesh(core_axis_name="c", subcore_axis_name="s")
    f32 = jnp.float32
    n_max = max(n0, n1)
    CH = _pick_ch(min(e0, e1) // NSUB, n_max * HALF, 34)

    @functools.partial(
        pl.kernel, mesh=mesh,
        compiler_params=pltpu.CompilerParams(use_tc_tiling_on_sc=False),
        out_type=[jax.ShapeDtypeStruct((n0, HALF), f32),
                  jax.ShapeDtypeStruct((n1, HALF), f32)],
        scratch_types=[
            pltpu.VMEM((CH,), jnp.int32), pltpu.VMEM((CH,), jnp.int32),
            pltpu.VMEM((CH, HALF), f32),
            pltpu.VMEM_SHARED((n_max, HALF), f32),
        ] + [pltpu.SemaphoreType.DMA] * 4)
    def k(edges0, edges1, out0, out1, idx0, idx1, ones, acc,
          si0, ss0, si1, ss1):
        c = lax.axis_index("c")
        s = lax.axis_index("s")
        _fill_rows(ones, CH, 0.0)
        for nn in sorted({n0, n1}):
            _copy_rows(ones, acc, s * (nn // NSUB), nn // NSUB)
        plsc.subcore_barrier()
        _fill_rows(ones, CH, 1.0)

        bufs = ((idx0, si0, ss0), (idx1, si1, ss1))

        def run(edges, d_off, e):
            per = e // NSUB
            iters = per // CH
            pairs, tail = iters // 2, iters % 2
            ebase = s * per

            def start_idx(b, off):
                idx, si, _ = bufs[b]
                pltpu.make_async_copy(
                    edges.at[pl.ds(d_off + off, CH)], idx, si).start()

            def start_scatter(b, off):
                idx, si, ss = bufs[b]
                pltpu.make_async_copy(
                    edges.at[pl.ds(d_off + off, CH)], idx, si).wait()
                pltpu.make_async_copy(ones, acc.at[idx], ss).start(add=True)

            def wait_scatter(b):
                idx, _, ss = bufs[b]
                pltpu.make_async_copy(ones, acc.at[idx], ss).wait()

            def pair(g, _):
                off0 = ebase + (2 * g) * CH
                off1 = off0 + CH

                @pl.when(g > 0)
                def _():
                    wait_scatter(0)

                start_idx(0, off0)
                start_scatter(0, off0)

                @pl.when(g > 0)
                def _():
                    wait_scatter(1)

                start_idx(1, off1)
                start_scatter(1, off1)
                return 0

            if pairs:
                lax.fori_loop(0, pairs, pair, 0)
            if tail:
                off = ebase + 2 * pairs * CH
                if pairs:
                    wait_scatter(0)
                start_idx(0, off)
                start_scatter(0, off)
            if pairs or tail:
                wait_scatter(0)
            if pairs:
                wait_scatter(1)

        @pl.when(c == 0)
        def _():
            run(edges0, d_off0, e0)

        @pl.when(c == 1)
        def _():
            run(edges1, d_off1, e1)

        plsc.subcore_barrier()

        @pl.when(c == 0)
        def _():
            rt = n0 // NSUB
            pltpu.sync_copy(acc.at[pl.ds(s * rt, rt)],
                            out0.at[pl.ds(s * rt, rt)])

        @pl.when(c == 1)
        def _():
            rt = n1 // NSUB
            pltpu.sync_copy(acc.at[pl.ds(s * rt, rt)],
                            out1.at[pl.ds(s * rt, rt)])

    return k


def _dotf(a, b):
    return jnp.dot(a, b, preferred_element_type=jnp.float32)


def _enc_body(x, wa, wb, ba, bb, w2ll, w2hl, w2lh, w2hh, b2a, b2b, lo, hi):
    xv = x[...]
    h_lo = jnp.maximum(_dotf(xv, wa[...]) + ba[...], 0.0)
    h_hi = jnp.maximum(_dotf(xv, wb[...]) + bb[...], 0.0)
    y_lo = _dotf(h_lo, w2ll[...]) + _dotf(h_hi, w2hl[...]) + b2a[...]
    y_hi = _dotf(h_lo, w2lh[...]) + _dotf(h_hi, w2hh[...]) + b2b[...]
    lo[...] = jnp.maximum(y_lo, 0.0)
    hi[...] = jnp.maximum(y_hi, 0.0)


def _encode(x, br, grid, *ws):
    n = x.shape[0]
    f32 = jnp.float32
    full = lambda a: pl.BlockSpec(a.shape, lambda i: (0, 0))
    blk = pl.BlockSpec((br, 128), lambda i: (i, 0))
    return pl.pallas_call(
        _enc_body,
        grid=(grid,),
        in_specs=[blk] + [full(w) for w in ws],
        out_specs=[blk, blk],
        out_shape=[jax.ShapeDtypeStruct((n, 128), f32)] * 2,
    )(x, *ws)


def _gnn_body(tlo, thi, qlo, qhi, w0ll, w0hl, w0lh, w0hh, b0a, b0b,
              w1ll, w1hl, w1lh, w1hh, b1a, b1b, olo, ohi):
    ql = qlo[...] * 0.5
    qh = qhi[...] * 0.5
    x_lo = tlo[...] + ql
    x_hi = thi[...] + qh
    h_lo = jnp.maximum(_dotf(x_lo, w0ll[...]) + _dotf(x_hi, w0hl[...])
                       + b0a[...], 0.0)
    h_hi = jnp.maximum(_dotf(x_lo, w0lh[...]) + _dotf(x_hi, w0hh[...])
                       + b0b[...], 0.0)
    g_lo = h_lo + ql
    g_hi = h_hi + qh
    olo[...] = jnp.maximum(_dotf(g_lo, w1ll[...]) + _dotf(g_hi, w1hl[...])
                           + b1a[...], 0.0)
    ohi[...] = jnp.maximum(_dotf(g_lo, w1lh[...]) + _dotf(g_hi, w1hh[...])
                           + b1b[...], 0.0)


def _gnn(tlo, thi, qlo, qhi, *ws):
    n = tlo.shape[0]
    f32 = jnp.float32
    full = lambda a: pl.BlockSpec(a.shape, lambda i: (0, 0))
    blk = pl.BlockSpec((1600, 128), lambda i: (i, 0))
    return pl.pallas_call(
        _gnn_body,
        grid=(n // 1600,),
        in_specs=[blk] * 4 + [full(w) for w in ws],
        out_specs=[blk, blk],
        out_shape=[jax.ShapeDtypeStruct((n, 128), f32)] * 2,
    )(tlo, thi, qlo, qhi, *ws)


def _upd_body(nvalid, br, hlo, hhi, mlo, mhi, clo, chi, olo, ohi, slo, shi):
    h2_lo = hlo[...] + (mlo[...] / jnp.maximum(clo[...], 1.0)) * 0.3
    h2_hi = hhi[...] + (mhi[...] / jnp.maximum(chi[...], 1.0)) * 0.3
    olo[...] = h2_lo
    ohi[...] = h2_hi
    i = pl.program_id(0)

    @pl.when(i == 0)
    def _():
        slo[...] = jnp.zeros_like(slo)
        shi[...] = jnp.zeros_like(shi)

    rows = lax.broadcasted_iota(jnp.int32, (br, 128), 0) + i * br
    mask = rows < nvalid
    slo[...] += jnp.sum(jnp.where(mask, h2_lo, 0.0), axis=0, keepdims=True)
    shi[...] += jnp.sum(jnp.where(mask, h2_hi, 0.0), axis=0, keepdims=True)


def _update(hlo, hhi, mlo, mhi, clo, chi, nvalid, br):
    n = hlo.shape[0]
    f32 = jnp.float32
    blk = pl.BlockSpec((br, 128), lambda i: (i, 0))
    one = pl.BlockSpec((1, 128), lambda i: (0, 0))
    return pl.pallas_call(
        functools.partial(_upd_body, nvalid, br),
        grid=(n // br,),
        in_specs=[blk] * 6,
        out_specs=[blk, blk, one, one],
        out_shape=[jax.ShapeDtypeStruct((n, 128), f32)] * 2
        + [jax.ShapeDtypeStruct((1, 128), f32)] * 2,
    )(hlo, hhi, mlo, mhi, clo, chi)


def _msum_body(mlo, mhi, clo, chi, slo, shi):
    t_lo = (mlo[...] / jnp.maximum(clo[...], 1.0)) * 0.3
    t_hi = (mhi[...] / jnp.maximum(chi[...], 1.0)) * 0.3
    i = pl.program_id(0)

    @pl.when(i == 0)
    def _():
        slo[...] = jnp.zeros_like(slo)
        shi[...] = jnp.zeros_like(shi)

    slo[...] += jnp.sum(t_lo, axis=0, keepdims=True)
    shi[...] += jnp.sum(t_hi, axis=0, keepdims=True)


def _msum(mlo, mhi, clo, chi, br):
    n = mlo.shape[0]
    f32 = jnp.float32
    blk = pl.BlockSpec((br, 128), lambda i: (i, 0))
    one = pl.BlockSpec((1, 128), lambda i: (0, 0))
    return pl.pallas_call(
        _msum_body,
        grid=(n // br,),
        in_specs=[blk] * 4,
        out_specs=[one, one],
        out_shape=[jax.ShapeDtypeStruct((1, 128), f32)] * 2,
    )(mlo, mhi, clo, chi)


def _fold4(p):
    return (p[:, 0:32] + p[:, 32:64] + p[:, 64:96] + p[:, 96:128])


def _final_body(omlo, omhi, oclo, ochi, hslo, hshi, tslo, tshi, eslo, eshi,
                ta_w, ta_b, ea_w, ea_b, ow1, ob1, ow2, ob2, out):
    t_lo = jnp.sum((omlo[...] / jnp.maximum(oclo[...], 1.0)) * 0.3,
                   axis=0, keepdims=True)
    t_hi = jnp.sum((omhi[...] / jnp.maximum(ochi[...], 1.0)) * 0.3,
                   axis=0, keepdims=True)
    hsum = jnp.concatenate(
        [_fold4(hslo[...] + tslo[...]), _fold4(hshi[...] + tshi[...])],
        axis=1)
    esum = jnp.concatenate(
        [_fold4(eslo[...] + t_lo), _fold4(eshi[...] + t_hi)], axis=1)
    hmean = hsum / N_TASK
    emean = esum / N_EDGE
    t_agg = jnp.maximum(_dotf(hmean, ta_w[...]) + ta_b[...], 0.0)
    e_agg = jnp.maximum(_dotf(emean, ea_w[...]) + ea_b[...], 0.0)
    comb = jnp.concatenate([t_agg, e_agg], axis=1)
    y = jnp.maximum(_dotf(comb, ow1[...]) + ob1[...], 0.0)
    out[...] = _dotf(y, ow2[...]) + ob2[...]


def _final(*args):
    f32 = jnp.float32
    full = lambda a: pl.BlockSpec(a.shape, lambda: (0, 0))
    return pl.pallas_call(
        _final_body,
        in_specs=[full(a) for a in args],
        out_specs=full(jnp.zeros((1, HID))),
        out_shape=jax.ShapeDtypeStruct((1, HID), f32),
    )(*args)


def _blk4(w):
    return jnp.kron(jnp.eye(4, dtype=jnp.float32), w)


def _b4(b):
    return jnp.tile(b, 4).reshape(1, 128)


def _wsplit(w):
    return (_blk4(w[:HALF, :HALF]), _blk4(w[HALF:, :HALF]),
            _blk4(w[:HALF, HALF:]), _blk4(w[HALF:, HALF:]))


def _pk(a):
    return a.reshape(-1, 128)


def _unpk(a):
    return a.reshape(-1, HALF)


def kernel(task_features, edge_features, queue_edges, type_edges,
           affinity_edges, topology_edges,
           te_w1, te_b1, te_w2, te_b2, ee_w1, ee_b1, ee_w2, ee_b2,
           gnn_w0, gnn_b0, gnn_w1, gnn_b1, ta_w, ta_b, ea_w, ea_b,
           out_w1, out_b1, out_w2, out_b2):
    r1 = lambda b: b.reshape(1, -1)
    xt = _pk(jnp.pad(task_features, ((0, NT_P - N_TASK), (0, HALF - 6))))
    xe = _pk(jnp.pad(edge_features, ((0, NE_P - N_EDGE), (0, HALF - 6))))
    qe = jnp.ravel(queue_edges)
    te = jnp.ravel(type_edges)
    ae = jnp.ravel(affinity_edges)
    oe = jnp.ravel(topology_edges)

    def enc_ws(w1, b1, w2, b2):
        w1e = jnp.pad(w1, ((0, HALF - w1.shape[0]), (0, 0)))
        return (_blk4(w1e[:, :HALF]), _blk4(w1e[:, HALF:]),
                _b4(b1[:HALF]), _b4(b1[HALF:]),
                *_wsplit(w2), _b4(b2[:HALF]), _b4(b2[HALF:]))

    tcnt_lo, acnt_lo = _make_ones_scatter(NT_P, EQ, EQ, NT_P, EA, 0)(te, ae)
    ecnt_lo, ocnt_lo = _make_ones_scatter(NE_P, EA, EA, NE_P, EA, EA)(ae, oe)
    tcnt = _pk(tcnt_lo)
    acnt = _pk(acnt_lo)
    ecnt = _pk(ecnt_lo)
    ocnt = _pk(ocnt_lo)

    t_lo, t_hi = _encode(xt, 1600, NT_P // 4 // 1600,
                         *enc_ws(te_w1, te_b1, te_w2, te_b2))
    e_lo, e_hi = _encode(xe, 2560, 1, *enc_ws(ee_w1, ee_b1, ee_w2, ee_b2))

    q_lo, q_hi = _make_segsum(NT_P, NT_P, EQ)(_unpk(t_lo), _unpk(t_hi), qe)
    h_lo, h_hi = _gnn(t_lo, t_hi, _pk(q_lo), _pk(q_hi),
                      *_wsplit(gnn_w0), _b4(gnn_b0[:HALF]), _b4(gnn_b0[HALF:]),
                      *_wsplit(gnn_w1), _b4(gnn_b1[:HALF]), _b4(gnn_b1[HALF:]))

    tm_lo, tm_hi = _make_segsum(NT_P, NT_P, EQ)(_unpk(h_lo), _unpk(h_hi), te)
    h2_lo, h2_hi, hs_lo, hs_hi = _update(
        h_lo, h_hi, _pk(tm_lo), _pk(tm_hi), tcnt, tcnt, N_TASK // 4, 1600)

    am_lo, am_hi = _make_segsum(NE_P, NT_P, EA, src_first=False)(
        _unpk(e_lo), _unpk(e_hi), ae)
    em_lo, em_hi = _make_segsum(NT_P, NE_P, EA)(
        _unpk(h2_lo), _unpk(h2_hi), ae)
    ts_lo, ts_hi = _msum(_pk(am_lo), _pk(am_hi), acnt, acnt, 1600)
    e2_lo, e2_hi, es_lo, es_hi = _update(
        e_lo, e_hi, _pk(em_lo), _pk(em_hi), ecnt, ecnt, N_EDGE // 4, 2560)

    om_lo, om_hi = _make_segsum(NE_P, NE_P, EA)(
        _unpk(e2_lo), _unpk(e2_hi), oe)

    out = _final(_pk(om_lo), _pk(om_hi), ocnt, ocnt,
                 hs_lo, hs_hi, ts_lo, ts_hi, es_lo, es_hi,
                 ta_w, r1(ta_b), ea_w, r1(ea_b),
                 out_w1, r1(out_b1), out_w2, r1(out_b2))
    return out.reshape(HID)
